# baseline (speedup 1.0000x reference)
import functools

import jax
import jax.numpy as jnp
from jax import lax
from jax.experimental import pallas as pl
from jax.experimental.pallas import tpu as pltpu
from jax.experimental.pallas import tpu_sc as plsc

B, N, K, H, FF = 1, 10000, 32, 128, 512
R = N * K
SCALE = 30.0
EPS = 1e-5

_NC, _NS = 2, 16
_NW = _NC * _NS
_PW = R // _NW
_NB = 8


def _sc_gather(table, idx2, pw, ch, nf, tailn):
    quads = nf // _NB
    rem = nf - quads * _NB
    mesh = plsc.VectorSubcoreMesh(core_axis_name="c", subcore_axis_name="s")

    @functools.partial(
        pl.kernel,
        out_type=jax.ShapeDtypeStruct((_NW * pw, H), jnp.float32),
        mesh=mesh,
        scratch_types=[
            pltpu.VMEM((pw,), jnp.int32),
            pltpu.VMEM((_NB, ch, H), jnp.float32),
        ] + [pltpu.SemaphoreType.DMA] * (2 * _NB),
    )
    def gk(table_hbm, idx_hbm, out_hbm, idx_all, rows, *sems):
        sg, ss = sems[:_NB], sems[_NB:]
        wid = lax.axis_index("s") * _NC + lax.axis_index("c")
        base = wid * pw
        pltpu.sync_copy(idx_hbm.at[wid], idx_all)

        def gather_chunk(c, b):
            return pltpu.async_copy(
                table_hbm.at[idx_all.at[pl.ds(c * ch, ch)]], rows.at[b], sg[b])

        def body(i, carry):
            c0 = _NB * i

            @pl.when(i > 0)
            def _drain():
                for b in range(_NB):
                    pltpu.make_async_copy(
                        rows.at[b], out_hbm.at[pl.ds(0, ch)], ss[b]).wait()

            gs = [gather_chunk(c0 + b, b) for b in range(_NB)]
            for b in range(_NB):
                gs[b].wait()
                pltpu.async_copy(
                    rows.at[b],
                    out_hbm.at[pl.ds(base + (c0 + b) * ch, ch)], ss[b])
            return carry

        lax.fori_loop(0, quads, body, 0)
        for b in range(_NB):
            pltpu.make_async_copy(
                rows.at[b], out_hbm.at[pl.ds(0, ch)], ss[b]).wait()
        gs = [gather_chunk(quads * _NB + j, j) for j in range(rem)]
        for j in range(rem):
            gs[j].wait()
            c = quads * _NB + j
            pltpu.sync_copy(rows.at[j], out_hbm.at[pl.ds(base + c * ch, ch)])
        if tailn:
            pltpu.async_copy(
                table_hbm.at[idx_all.at[pl.ds(nf * ch, tailn)]],
                rows.at[rem, pl.ds(0, tailn)], sg[rem]).wait()
            pltpu.sync_copy(rows.at[rem, pl.ds(0, tailn)],
                            out_hbm.at[pl.ds(base + nf * ch, tailn)])

    return gk(table, idx2)


_TN = 400
_TA = 400


def _gelu(x):
    return 0.5 * x * (1.0 + lax.erf(x * 0.7071067811865476))


def _ln(x, g, b):
    m = jnp.mean(x, axis=-1, keepdims=True)
    xc = x - m
    v = jnp.mean(xc * xc, axis=-1, keepdims=True)
    return xc * lax.rsqrt(v + EPS) * g + b


def _dot(a, b):
    return jnp.dot(a.astype(jnp.bfloat16), b.astype(jnp.bfloat16),
                   preferred_element_type=jnp.float32)


def _b(x):
    return x.astype(jnp.bfloat16)


def _ln_mxu(x, g, b, one):
    xm = jnp.dot(x, one, preferred_element_type=jnp.float32)
    xc = x - xm
    v = jnp.dot(xc * xc, one, preferred_element_type=jnp.float32)
    return xc * lax.rsqrt(v + EPS) * g + b


def _stage_a_body(hv_ref, he_ref, gg_ref, w1a, w1b, w1c, b1r, w2, b2r, w3,
                  b3r, wi, bir, wo, bor, g1r, be1r, g2r, be2r, out_ref):
    hv = hv_ref[...]
    he = he_ref[...]
    gg = gg_ref[...]
    tv = _b(_dot(hv, w1a[...]) + b1r[...])
    z = _b(_dot(he, w1b[...])) + _b(_dot(gg, w1c[...]))
    z = z.reshape(_TA, K, H) + tv[:, None, :]
    m1 = _gelu(z).reshape(_TA * K, H)
    m2 = _gelu(_b(_dot(m1, w2[...])) + _b(b2r[...]))
    m2s = jnp.sum(m2.reshape(_TA, K, H), axis=1)
    dh = (_dot(m2s, w3[...]) + K * b3r[...]) * (1.0 / SCALE)
    x = _ln(hv + dh, g1r[...], be1r[...])
    f = _gelu(_b(_dot(x, wi[...])) + _b(bir[...]))
    x2 = x + _dot(f, wo[...]) + bor[...]
    out_ref[...] = _ln(x2, g2r[...], be2r[...])


def _stage_c_body(hv_ref, he_ref, gg_ref, w1a, w1b, w1c, b1r, w2, b2r, w3,
                  b3r, g3r, be3r, one_r, out_ref):
    hv = hv_ref[...]
    he = he_ref[...]
    gg = gg_ref[...]
    tv = _b(_dot(hv, w1a[...]) + b1r[...])
    z = _b(_dot(he, w1b[...])) + _b(_dot(gg, w1c[...]))
    z = z.reshape(_TN, K, H) + tv[:, None, :]
    m1 = _gelu(z).reshape(_TN * K, H)
    m2 = _gelu(_b(_dot(m1, w2[...])) + _b(b2r[...]))
    m3 = _dot(m2, w3[...]) + b3r[...]
    out_ref[...] = _ln_mxu(he + m3, g3r[...], be3r[...], one_r[...])


def _node_spec():
    return pl.BlockSpec((_TN, H), lambda i: (i, 0))


def _edge_spec():
    return pl.BlockSpec((_TN * K, H), lambda i: (i, 0))


def _w_spec(r, c):
    return pl.BlockSpec((r, c), lambda i: (0, 0))


def _stage_a(hv, he2, gg, w1a, w1b, w1c, b1, w2, b2, w3, b3, wi, bi, wo, bo,
             g1, be1, g2, be2, n_nodes, off):
    ob = off // _TA
    grid = (n_nodes // _TA,)
    in_specs = [
        pl.BlockSpec((_TA, H), lambda i: (i + ob, 0)),
        pl.BlockSpec((_TA * K, H), lambda i: (i + ob, 0)),
        pl.BlockSpec((_TA * K, H), lambda i: (i, 0)),
        _w_spec(H, H), _w_spec(H, H), _w_spec(H, H), _w_spec(1, H),
        _w_spec(H, H), _w_spec(1, H), _w_spec(H, H), _w_spec(1, H),
        _w_spec(H, FF), _w_spec(1, FF), _w_spec(FF, H), _w_spec(1, H),
        _w_spec(1, H), _w_spec(1, H), _w_spec(1, H), _w_spec(1, H),
    ]
    return pl.pallas_call(
        _stage_a_body,
        grid=grid,
        in_specs=in_specs,
        out_specs=pl.BlockSpec((_TA, H), lambda i: (i, 0)),
        out_shape=jax.ShapeDtypeStruct((n_nodes, H), jnp.float32),
    )(hv, he2, gg, w1a, w1b, w1c, b1, w2, b2, w3, b3, wi, bi, wo, bo,
      g1, be1, g2, be2)


def _stage_c(hv2, he2, gg, w1a, w1b, w1c, b1, w2, b2, w3, b3, g3, be3, one):
    grid = (N // _TN,)
    in_specs = [
        _node_spec(), _edge_spec(), _edge_spec(),
        _w_spec(H, H), _w_spec(H, H), _w_spec(H, H), _w_spec(1, H),
        _w_spec(H, H), _w_spec(1, H), _w_spec(H, H), _w_spec(1, H),
        _w_spec(1, H), _w_spec(1, H), _w_spec(H, H),
    ]
    return pl.pallas_call(
        _stage_c_body,
        grid=grid,
        in_specs=in_specs,
        out_specs=_edge_spec(),
        out_shape=jax.ShapeDtypeStruct((R, H), jnp.float32),
    )(hv2, he2, gg, w1a, w1b, w1c, b1, w2, b2, w3, b3, g3, be3, one)


def kernel(h_V, h_E, E_idx, W1, b1, W2, b2, W3, b3, W11, b11, W12, b12,
           W13, b13, Wi, bi, Wo, bo, g1, be1, g2, be2, g3, be3):
    hv = h_V[0]
    he2 = h_E[0].reshape(R, H)
    idx_full = E_idx[0].reshape(_NW, R // _NW)

    row = lambda v: v.reshape(1, -1)
    wa = (W1[:H], W1[H:2 * H], W1[2 * H:], row(b1),
          W2, row(b2), W3, row(b3),
          Wi, row(bi), Wo, row(bo),
          row(g1), row(be1), row(g2), row(be2))

    g1v = _sc_gather(hv, idx_full, R // _NW, 112, 89, 32)
    hv2 = _stage_a(hv, he2, g1v, *wa, N, 0)
    g2v = _sc_gather(hv2, idx_full, R // _NW, 112, 89, 32)
    one = jnp.full((H, H), 1.0 / H, jnp.float32)
    he_out = _stage_c(hv2, he2, g2v,
                      W11[:H], W11[H:2 * H], W11[2 * H:], row(b11),
                      W12, row(b12), W13, row(b13),
                      row(g3), row(be3), one)
    return hv2[None], he_out.reshape(B, N, K, H)


# speedup vs baseline: 1.0040x; 1.0040x over previous
import functools

import jax
import jax.numpy as jnp
from jax import lax
from jax.experimental import pallas as pl
from jax.experimental.pallas import tpu as pltpu
from jax.experimental.pallas import tpu_sc as plsc

B, N, K, H, FF = 1, 10000, 32, 128, 512
R = N * K
SCALE = 30.0
EPS = 1e-5

_NC, _NS = 2, 16
_NW = _NC * _NS
_PW = R // _NW
_NB = 8


def _sc_gather(table, idx2, pw, ch, nf, tailn):
    quads = nf // _NB
    rem = nf - quads * _NB
    mesh = plsc.VectorSubcoreMesh(core_axis_name="c", subcore_axis_name="s")

    @functools.partial(
        pl.kernel,
        out_type=jax.ShapeDtypeStruct((_NW * pw, H), jnp.float32),
        mesh=mesh,
        scratch_types=[
            pltpu.VMEM((pw,), jnp.int32),
            pltpu.VMEM((_NB, ch, H), jnp.float32),
        ] + [pltpu.SemaphoreType.DMA] * (2 * _NB),
    )
    def gk(table_hbm, idx_hbm, out_hbm, idx_all, rows, *sems):
        sg, ss = sems[:_NB], sems[_NB:]
        wid = lax.axis_index("s") * _NC + lax.axis_index("c")
        base = wid * pw
        pltpu.sync_copy(idx_hbm.at[wid], idx_all)

        def gather_chunk(c, b):
            return pltpu.async_copy(
                table_hbm.at[idx_all.at[pl.ds(c * ch, ch)]], rows.at[b], sg[b])

        def body(i, carry):
            c0 = _NB * i

            gs = []
            for b in range(_NB):
                @pl.when(i > 0)
                def _drain(b=b):
                    pltpu.make_async_copy(
                        rows.at[b], out_hbm.at[pl.ds(0, ch)], ss[b]).wait()
                gs.append(gather_chunk(c0 + b, b))
            for b in range(_NB):
                gs[b].wait()
                pltpu.async_copy(
                    rows.at[b],
                    out_hbm.at[pl.ds(base + (c0 + b) * ch, ch)], ss[b])
            return carry

        lax.fori_loop(0, quads, body, 0)
        for b in range(_NB):
            pltpu.make_async_copy(
                rows.at[b], out_hbm.at[pl.ds(0, ch)], ss[b]).wait()
        gs = [gather_chunk(quads * _NB + j, j) for j in range(rem)]
        for j in range(rem):
            gs[j].wait()
            c = quads * _NB + j
            pltpu.sync_copy(rows.at[j], out_hbm.at[pl.ds(base + c * ch, ch)])
        if tailn:
            pltpu.async_copy(
                table_hbm.at[idx_all.at[pl.ds(nf * ch, tailn)]],
                rows.at[rem, pl.ds(0, tailn)], sg[rem]).wait()
            pltpu.sync_copy(rows.at[rem, pl.ds(0, tailn)],
                            out_hbm.at[pl.ds(base + nf * ch, tailn)])

    return gk(table, idx2)


_TN = 400
_TA = 400


def _gelu(x):
    return 0.5 * x * (1.0 + lax.erf(x * 0.7071067811865476))


def _ln(x, g, b):
    m = jnp.mean(x, axis=-1, keepdims=True)
    xc = x - m
    v = jnp.mean(xc * xc, axis=-1, keepdims=True)
    return xc * lax.rsqrt(v + EPS) * g + b


def _dot(a, b):
    return jnp.dot(a.astype(jnp.bfloat16), b.astype(jnp.bfloat16),
                   preferred_element_type=jnp.float32)


def _b(x):
    return x.astype(jnp.bfloat16)


def _ln_mxu(x, g, b, one):
    xm = jnp.dot(x, one, preferred_element_type=jnp.float32)
    xc = x - xm
    v = jnp.dot(xc * xc, one, preferred_element_type=jnp.float32)
    return xc * lax.rsqrt(v + EPS) * g + b


def _stage_a_body(hv_ref, he_ref, gg_ref, w1a, w1b, w1c, b1r, w2, b2r, w3,
                  b3r, wi, bir, wo, bor, g1r, be1r, g2r, be2r, out_ref):
    hv = hv_ref[...]
    he = he_ref[...]
    gg = gg_ref[...]
    tv = _b(_dot(hv, w1a[...]) + b1r[...])
    z = _b(_dot(he, w1b[...])) + _b(_dot(gg, w1c[...]))
    z = z.reshape(_TA, K, H) + tv[:, None, :]
    m1 = _gelu(z).reshape(_TA * K, H)
    m2 = _gelu(_b(_dot(m1, w2[...])) + _b(b2r[...]))
    m2s = jnp.sum(m2.reshape(_TA, K, H), axis=1)
    dh = (_dot(m2s, w3[...]) + K * b3r[...]) * (1.0 / SCALE)
    x = _ln(hv + dh, g1r[...], be1r[...])
    f = _gelu(_b(_dot(x, wi[...])) + _b(bir[...]))
    x2 = x + _dot(f, wo[...]) + bor[...]
    out_ref[...] = _ln(x2, g2r[...], be2r[...])


def _stage_c_body(hv_ref, he_ref, gg_ref, w1a, w1b, w1c, b1r, w2, b2r, w3,
                  b3r, g3r, be3r, one_r, out_ref):
    hv = hv_ref[...]
    he = he_ref[...]
    gg = gg_ref[...]
    tv = _b(_dot(hv, w1a[...]) + b1r[...])
    z = _b(_dot(he, w1b[...])) + _b(_dot(gg, w1c[...]))
    z = z.reshape(_TN, K, H) + tv[:, None, :]
    m1 = _gelu(z).reshape(_TN * K, H)
    m2 = _gelu(_b(_dot(m1, w2[...])) + _b(b2r[...]))
    m3 = _dot(m2, w3[...]) + b3r[...]
    out_ref[...] = _ln_mxu(he + m3, g3r[...], be3r[...], one_r[...])


def _node_spec():
    return pl.BlockSpec((_TN, H), lambda i: (i, 0))


def _edge_spec():
    return pl.BlockSpec((_TN * K, H), lambda i: (i, 0))


def _w_spec(r, c):
    return pl.BlockSpec((r, c), lambda i: (0, 0))


def _stage_a(hv, he2, gg, w1a, w1b, w1c, b1, w2, b2, w3, b3, wi, bi, wo, bo,
             g1, be1, g2, be2, n_nodes, off):
    ob = off // _TA
    grid = (n_nodes // _TA,)
    in_specs = [
        pl.BlockSpec((_TA, H), lambda i: (i + ob, 0)),
        pl.BlockSpec((_TA * K, H), lambda i: (i + ob, 0)),
        pl.BlockSpec((_TA * K, H), lambda i: (i, 0)),
        _w_spec(H, H), _w_spec(H, H), _w_spec(H, H), _w_spec(1, H),
        _w_spec(H, H), _w_spec(1, H), _w_spec(H, H), _w_spec(1, H),
        _w_spec(H, FF), _w_spec(1, FF), _w_spec(FF, H), _w_spec(1, H),
        _w_spec(1, H), _w_spec(1, H), _w_spec(1, H), _w_spec(1, H),
    ]
    return pl.pallas_call(
        _stage_a_body,
        grid=grid,
        in_specs=in_specs,
        out_specs=pl.BlockSpec((_TA, H), lambda i: (i, 0)),
        out_shape=jax.ShapeDtypeStruct((n_nodes, H), jnp.float32),
    )(hv, he2, gg, w1a, w1b, w1c, b1, w2, b2, w3, b3, wi, bi, wo, bo,
      g1, be1, g2, be2)


def _stage_c(hv2, he2, gg, w1a, w1b, w1c, b1, w2, b2, w3, b3, g3, be3, one):
    grid = (N // _TN,)
    in_specs = [
        _node_spec(), _edge_spec(), _edge_spec(),
        _w_spec(H, H), _w_spec(H, H), _w_spec(H, H), _w_spec(1, H),
        _w_spec(H, H), _w_spec(1, H), _w_spec(H, H), _w_spec(1, H),
        _w_spec(1, H), _w_spec(1, H), _w_spec(H, H),
    ]
    return pl.pallas_call(
        _stage_c_body,
        grid=grid,
        in_specs=in_specs,
        out_specs=_edge_spec(),
        out_shape=jax.ShapeDtypeStruct((R, H), jnp.float32),
    )(hv2, he2, gg, w1a, w1b, w1c, b1, w2, b2, w3, b3, g3, be3, one)


def kernel(h_V, h_E, E_idx, W1, b1, W2, b2, W3, b3, W11, b11, W12, b12,
           W13, b13, Wi, bi, Wo, bo, g1, be1, g2, be2, g3, be3):
    hv = h_V[0]
    he2 = h_E[0].reshape(R, H)
    idx_full = E_idx[0].reshape(_NW, R // _NW)

    row = lambda v: v.reshape(1, -1)
    wa = (W1[:H], W1[H:2 * H], W1[2 * H:], row(b1),
          W2, row(b2), W3, row(b3),
          Wi, row(bi), Wo, row(bo),
          row(g1), row(be1), row(g2), row(be2))

    g1v = _sc_gather(hv, idx_full, R // _NW, 80, 125, 0)
    hv2 = _stage_a(hv, he2, g1v, *wa, N, 0)
    g2v = _sc_gather(hv2, idx_full, R // _NW, 80, 125, 0)
    one = jnp.full((H, H), 1.0 / H, jnp.float32)
    he_out = _stage_c(hv2, he2, g2v,
                      W11[:H], W11[H:2 * H], W11[2 * H:], row(b11),
                      W12, row(b12), W13, row(b13),
                      row(g3), row(be3), one)
    return hv2[None], he_out.reshape(B, N, K, H)


# speedup vs baseline: 1.0111x; 1.0071x over previous
import functools

import jax
import jax.numpy as jnp
from jax import lax
from jax.experimental import pallas as pl
from jax.experimental.pallas import tpu as pltpu
from jax.experimental.pallas import tpu_sc as plsc

B, N, K, H, FF = 1, 10000, 32, 128, 512
R = N * K
SCALE = 30.0
EPS = 1e-5

_NC, _NS = 2, 16
_NW = _NC * _NS
_PW = R // _NW
_NB = 10


def _sc_gather(table, idx2, pw, ch, nf, tailn):
    quads = nf // _NB
    rem = nf - quads * _NB
    mesh = plsc.VectorSubcoreMesh(core_axis_name="c", subcore_axis_name="s")

    @functools.partial(
        pl.kernel,
        out_type=jax.ShapeDtypeStruct((_NW * pw, H), jnp.float32),
        mesh=mesh,
        scratch_types=[
            pltpu.VMEM((pw,), jnp.int32),
            pltpu.VMEM((_NB, ch, H), jnp.float32),
        ] + [pltpu.SemaphoreType.DMA] * (2 * _NB),
    )
    def gk(table_hbm, idx_hbm, out_hbm, idx_all, rows, *sems):
        sg, ss = sems[:_NB], sems[_NB:]
        wid = lax.axis_index("s") * _NC + lax.axis_index("c")
        base = wid * pw
        pltpu.sync_copy(idx_hbm.at[wid], idx_all)

        def gather_chunk(c, b):
            return pltpu.async_copy(
                table_hbm.at[idx_all.at[pl.ds(c * ch, ch)]], rows.at[b], sg[b])

        def body(i, carry):
            c0 = _NB * i

            @pl.when(i > 0)
            def _drain():
                for b in range(_NB):
                    pltpu.make_async_copy(
                        rows.at[b], out_hbm.at[pl.ds(0, ch)], ss[b]).wait()

            gs = [gather_chunk(c0 + b, b) for b in range(_NB)]
            for b in range(_NB):
                gs[b].wait()
                pltpu.async_copy(
                    rows.at[b],
                    out_hbm.at[pl.ds(base + (c0 + b) * ch, ch)], ss[b])
            return carry

        lax.fori_loop(0, quads, body, 0)
        for b in range(_NB):
            pltpu.make_async_copy(
                rows.at[b], out_hbm.at[pl.ds(0, ch)], ss[b]).wait()
        gs = [gather_chunk(quads * _NB + j, j) for j in range(rem)]
        for j in range(rem):
            gs[j].wait()
            c = quads * _NB + j
            pltpu.sync_copy(rows.at[j], out_hbm.at[pl.ds(base + c * ch, ch)])
        if tailn:
            pltpu.async_copy(
                table_hbm.at[idx_all.at[pl.ds(nf * ch, tailn)]],
                rows.at[rem, pl.ds(0, tailn)], sg[rem]).wait()
            pltpu.sync_copy(rows.at[rem, pl.ds(0, tailn)],
                            out_hbm.at[pl.ds(base + nf * ch, tailn)])

    return gk(table, idx2)


_TN = 400
_TA = 400


def _gelu(x):
    return 0.5 * x * (1.0 + lax.erf(x * 0.7071067811865476))


def _ln(x, g, b):
    m = jnp.mean(x, axis=-1, keepdims=True)
    xc = x - m
    v = jnp.mean(xc * xc, axis=-1, keepdims=True)
    return xc * lax.rsqrt(v + EPS) * g + b


def _dot(a, b):
    return jnp.dot(a.astype(jnp.bfloat16), b.astype(jnp.bfloat16),
                   preferred_element_type=jnp.float32)


def _b(x):
    return x.astype(jnp.bfloat16)


def _ln_mxu(x, g, b, one):
    xm = jnp.dot(x, one, preferred_element_type=jnp.float32)
    xc = x - xm
    v = jnp.dot(xc * xc, one, preferred_element_type=jnp.float32)
    return xc * lax.rsqrt(v + EPS) * g + b


def _stage_a_body(hv_ref, he_ref, gg_ref, w1a, w1b, w1c, b1r, w2, b2r, w3,
                  b3r, wi, bir, wo, bor, g1r, be1r, g2r, be2r, out_ref):
    hv = hv_ref[...]
    he = he_ref[...]
    gg = gg_ref[...]
    tv = _b(_dot(hv, w1a[...]) + b1r[...])
    z = _b(_dot(he, w1b[...])) + _b(_dot(gg, w1c[...]))
    z = z.reshape(_TA, K, H) + tv[:, None, :]
    m1 = _gelu(z).reshape(_TA * K, H)
    m2 = _gelu(_b(_dot(m1, w2[...])) + _b(b2r[...]))
    m2s = jnp.sum(m2.reshape(_TA, K, H), axis=1)
    dh = (_dot(m2s, w3[...]) + K * b3r[...]) * (1.0 / SCALE)
    x = _ln(hv + dh, g1r[...], be1r[...])
    f = _gelu(_b(_dot(x, wi[...])) + _b(bir[...]))
    x2 = x + _dot(f, wo[...]) + bor[...]
    out_ref[...] = _ln(x2, g2r[...], be2r[...])


def _stage_c_body(hv_ref, he_ref, gg_ref, w1a, w1b, w1c, b1r, w2, b2r, w3,
                  b3r, g3r, be3r, one_r, out_ref):
    hv = hv_ref[...]
    he = he_ref[...]
    gg = gg_ref[...]
    tv = _b(_dot(hv, w1a[...]) + b1r[...])
    z = _b(_dot(he, w1b[...])) + _b(_dot(gg, w1c[...]))
    z = z.reshape(_TN, K, H) + tv[:, None, :]
    m1 = _gelu(z).reshape(_TN * K, H)
    m2 = _gelu(_b(_dot(m1, w2[...])) + _b(b2r[...]))
    m3 = _dot(m2, w3[...]) + b3r[...]
    out_ref[...] = _ln_mxu(he + m3, g3r[...], be3r[...], one_r[...])


def _node_spec():
    return pl.BlockSpec((_TN, H), lambda i: (i, 0))


def _edge_spec():
    return pl.BlockSpec((_TN * K, H), lambda i: (i, 0))


def _w_spec(r, c):
    return pl.BlockSpec((r, c), lambda i: (0, 0))


def _stage_a(hv, he2, gg, w1a, w1b, w1c, b1, w2, b2, w3, b3, wi, bi, wo, bo,
             g1, be1, g2, be2, n_nodes, off):
    ob = off // _TA
    grid = (n_nodes // _TA,)
    in_specs = [
        pl.BlockSpec((_TA, H), lambda i: (i + ob, 0)),
        pl.BlockSpec((_TA * K, H), lambda i: (i + ob, 0)),
        pl.BlockSpec((_TA * K, H), lambda i: (i, 0)),
        _w_spec(H, H), _w_spec(H, H), _w_spec(H, H), _w_spec(1, H),
        _w_spec(H, H), _w_spec(1, H), _w_spec(H, H), _w_spec(1, H),
        _w_spec(H, FF), _w_spec(1, FF), _w_spec(FF, H), _w_spec(1, H),
        _w_spec(1, H), _w_spec(1, H), _w_spec(1, H), _w_spec(1, H),
    ]
    return pl.pallas_call(
        _stage_a_body,
        grid=grid,
        in_specs=in_specs,
        out_specs=pl.BlockSpec((_TA, H), lambda i: (i, 0)),
        out_shape=jax.ShapeDtypeStruct((n_nodes, H), jnp.float32),
    )(hv, he2, gg, w1a, w1b, w1c, b1, w2, b2, w3, b3, wi, bi, wo, bo,
      g1, be1, g2, be2)


def _stage_c(hv2, he2, gg, w1a, w1b, w1c, b1, w2, b2, w3, b3, g3, be3, one):
    grid = (N // _TN,)
    in_specs = [
        _node_spec(), _edge_spec(), _edge_spec(),
        _w_spec(H, H), _w_spec(H, H), _w_spec(H, H), _w_spec(1, H),
        _w_spec(H, H), _w_spec(1, H), _w_spec(H, H), _w_spec(1, H),
        _w_spec(1, H), _w_spec(1, H), _w_spec(H, H),
    ]
    return pl.pallas_call(
        _stage_c_body,
        grid=grid,
        in_specs=in_specs,
        out_specs=_edge_spec(),
        out_shape=jax.ShapeDtypeStruct((R, H), jnp.float32),
    )(hv2, he2, gg, w1a, w1b, w1c, b1, w2, b2, w3, b3, g3, be3, one)


def kernel(h_V, h_E, E_idx, W1, b1, W2, b2, W3, b3, W11, b11, W12, b12,
           W13, b13, Wi, bi, Wo, bo, g1, be1, g2, be2, g3, be3):
    hv = h_V[0]
    he2 = h_E[0].reshape(R, H)
    idx_full = E_idx[0].reshape(_NW, R // _NW)

    row = lambda v: v.reshape(1, -1)
    wa = (W1[:H], W1[H:2 * H], W1[2 * H:], row(b1),
          W2, row(b2), W3, row(b3),
          Wi, row(bi), Wo, row(bo),
          row(g1), row(be1), row(g2), row(be2))

    g1v = _sc_gather(hv, idx_full, R // _NW, 80, 125, 0)
    hv2 = _stage_a(hv, he2, g1v, *wa, N, 0)
    g2v = _sc_gather(hv2, idx_full, R // _NW, 80, 125, 0)
    one = jnp.full((H, H), 1.0 / H, jnp.float32)
    he_out = _stage_c(hv2, he2, g2v,
                      W11[:H], W11[H:2 * H], W11[2 * H:], row(b11),
                      W12, row(b12), W13, row(b13),
                      row(g3), row(be3), one)
    return hv2[None], he_out.reshape(B, N, K, H)
